# baseline (speedup 1.0000x reference)
import jax
import jax.numpy as jnp
from jax import lax
from jax.experimental import pallas as pl
from jax.experimental.pallas import tpu as pltpu
from jax.experimental.pallas import tpu_sc as plsc

VOCAB = 1_000_000
DIM = 64
B = 16384
K = 21
N = B * K
NW = 32
NPW = N // NW
CH = 512
SUB = 64
NSUB = CH // SUB
NCH = NPW // CH
L = 16


def _dot_chunk(ur, vr, wbuf, oc):
    def pair(p, _):
        s = None
        for d0 in range(0, DIM, L):
            t = ur[p, pl.ds(d0, L)] * vr[p, pl.ds(d0, L)]
            s = t if s is None else s + t
        wbuf[pl.ds(p * L, L)] = s
        return 0

    lax.fori_loop(0, CH, pair, 0, unroll=4)

    iot = lax.iota(jnp.int32, L)

    def group(g, _):
        ibase = (g * L + iot) * L
        acc = None
        for l in range(L):
            t = plsc.load_gather(wbuf, [ibase + l])
            acc = t if acc is None else acc + t
        oc[pl.ds(g * L, L)] = acc
        return 0

    lax.fori_loop(0, CH // L, group, 0)


def _sc_body(iu_hbm, iv_hbm, u_tab, v_tab, out_hbm,
             iu_v, iv_v, ur, vr, wbuf, oc, sem):
    cid = lax.axis_index("c")
    sid = lax.axis_index("s")
    wid = sid * 2 + cid

    def chunk(c, _):
        base = pl.multiple_of(wid * NPW + c * CH, CH)
        row0 = pl.multiple_of(base // SUB, NSUB)
        pltpu.sync_copy(iu_hbm.at[pl.ds(row0, NSUB)], iu_v)
        pltpu.sync_copy(iv_hbm.at[pl.ds(row0, NSUB)], iv_v)
        cps = []
        for j in range(NSUB):
            cps.append(pltpu.async_copy(
                u_tab.at[iu_v.at[j]], ur.at[pl.ds(j * SUB, SUB)], sem))
            cps.append(pltpu.async_copy(
                v_tab.at[iv_v.at[j]], vr.at[pl.ds(j * SUB, SUB)], sem))
        for cp in cps:
            cp.wait()
        _dot_chunk(ur, vr, wbuf, oc)
        pltpu.sync_copy(oc, out_hbm.at[pl.ds(base, CH)])
        return 0

    lax.fori_loop(0, NCH, chunk, 0)


def kernel(x, U, V):
    idx = x.astype(jnp.int32)
    iu = idx[:, :, 0].reshape(N // SUB, SUB)
    iv = idx[:, :, 1].reshape(N // SUB, SUB)
    mesh = plsc.VectorSubcoreMesh(core_axis_name="c", subcore_axis_name="s")
    f = pl.kernel(
        _sc_body,
        out_type=jax.ShapeDtypeStruct((N,), jnp.float32),
        mesh=mesh,
        compiler_params=pltpu.CompilerParams(
            use_tc_tiling_on_sc=False, needs_layout_passes=False),
        scratch_types=[
            pltpu.VMEM((NSUB, SUB), jnp.int32),
            pltpu.VMEM((NSUB, SUB), jnp.int32),
            pltpu.VMEM((CH, DIM), jnp.float32),
            pltpu.VMEM((CH, DIM), jnp.float32),
            pltpu.VMEM((CH * L,), jnp.float32),
            pltpu.VMEM((CH,), jnp.float32),
            pltpu.SemaphoreType.DMA,
        ],
    )
    return f(iu, iv, U, V).reshape(B, K)
